# baseline (speedup 1.0000x reference)
import jax
import jax.numpy as jnp
from jax.experimental import pallas as pl

B, N, S, D1, D2 = 4, 8192, 2048, 128, 256
C1, C2 = 256, 128
NB = 4096
NBLK = N // NB
M = B * N


def _k1(p1t_ref, p2_ref, f2t_ref, f1t_ref, w1at_ref, w1bt_ref, b1_ref,
        h1_ref, s_ref, ss_ref):
    b = pl.program_id(0)
    n = pl.program_id(1)
    p1 = p1t_ref[0]
    p2 = p2_ref[0]
    d = (jnp.sum(p1 * p1, axis=1, keepdims=True)
         - 2.0 * jnp.dot(p1, p2, preferred_element_type=jnp.float32)
         + jnp.sum(p2 * p2, axis=0, keepdims=True))
    iota = jax.lax.broadcasted_iota(jnp.int32, (NB, S), 1)

    def extract(dc):
        m = jnp.min(dc, axis=1, keepdims=True)
        i = jnp.min(jnp.where(dc <= m, iota, S), axis=1, keepdims=True)
        dn = jnp.where(iota == i, jnp.inf, dc)
        return m, i, dn

    m1, i1, d = extract(d)
    m2, i2, d = extract(d)
    m3, i3, _ = extract(d)
    r1 = 1.0 / (m1 + 1e-8)
    r2 = 1.0 / (m2 + 1e-8)
    r3 = 1.0 / (m3 + 1e-8)
    nrm = r1 + r2 + r3
    wm = (jnp.where(iota == i1, r1 / nrm, 0.0)
          + jnp.where(iota == i2, r2 / nrm, 0.0)
          + jnp.where(iota == i3, r3 / nrm, 0.0))
    interp = jnp.dot(wm, f2t_ref[0], preferred_element_type=jnp.float32)
    h1 = (jnp.dot(f1t_ref[0], w1at_ref[...], preferred_element_type=jnp.float32)
          + jnp.dot(interp, w1bt_ref[...], preferred_element_type=jnp.float32)
          + b1_ref[...])
    h1_ref[0] = h1

    @pl.when(jnp.logical_and(b == 0, n == 0))
    def _():
        s_ref[...] = jnp.zeros_like(s_ref)
        ss_ref[...] = jnp.zeros_like(ss_ref)

    s_ref[...] += jnp.sum(h1, axis=0, keepdims=True)
    ss_ref[...] += jnp.sum(h1 * h1, axis=0, keepdims=True)


def _k2(h1_ref, s1_ref, ss1_ref, g1_ref, be1_ref, w2t_ref, b2_ref,
        h2_ref, s_ref, ss_ref):
    b = pl.program_id(0)
    n = pl.program_id(1)
    mean = s1_ref[...] / M
    var = ss1_ref[...] / M - mean * mean
    scale = g1_ref[...] * jax.lax.rsqrt(var + 1e-5)
    z = jnp.maximum((h1_ref[0] - mean) * scale + be1_ref[...], 0.0)
    h2 = jnp.dot(z, w2t_ref[...], preferred_element_type=jnp.float32) + b2_ref[...]
    h2_ref[0] = h2

    @pl.when(jnp.logical_and(b == 0, n == 0))
    def _():
        s_ref[...] = jnp.zeros_like(s_ref)
        ss_ref[...] = jnp.zeros_like(ss_ref)

    s_ref[...] += jnp.sum(h2, axis=0, keepdims=True)
    ss_ref[...] += jnp.sum(h2 * h2, axis=0, keepdims=True)


def _k3(h2_ref, s2_ref, ss2_ref, g2_ref, be2_ref, out_ref):
    mean = s2_ref[...] / M
    var = ss2_ref[...] / M - mean * mean
    scale = g2_ref[...] * jax.lax.rsqrt(var + 1e-5)
    z = jnp.maximum((h2_ref[0] - mean) * scale + be2_ref[...], 0.0)
    out_ref[0] = z.T


def kernel(pos1, pos2, feats1, feats2, W1, b1, g1, beta1, W2, b2, g2, beta2):
    p1t = jnp.transpose(pos1, (0, 2, 1))
    f1t = jnp.transpose(feats1, (0, 2, 1))
    f2t = jnp.transpose(feats2, (0, 2, 1))
    w1at = jnp.transpose(W1[:, :D1])
    w1bt = jnp.transpose(W1[:, D1:])
    w2t = jnp.transpose(W2)

    h1, s1, ss1 = pl.pallas_call(
        _k1,
        grid=(B, NBLK),
        in_specs=[
            pl.BlockSpec((1, NB, 3), lambda b, n: (b, n, 0)),
            pl.BlockSpec((1, 3, S), lambda b, n: (b, 0, 0)),
            pl.BlockSpec((1, S, D2), lambda b, n: (b, 0, 0)),
            pl.BlockSpec((1, NB, D1), lambda b, n: (b, n, 0)),
            pl.BlockSpec((D1, C1), lambda b, n: (0, 0)),
            pl.BlockSpec((D2, C1), lambda b, n: (0, 0)),
            pl.BlockSpec((1, C1), lambda b, n: (0, 0)),
        ],
        out_specs=[
            pl.BlockSpec((1, NB, C1), lambda b, n: (b, n, 0)),
            pl.BlockSpec((1, C1), lambda b, n: (0, 0)),
            pl.BlockSpec((1, C1), lambda b, n: (0, 0)),
        ],
        out_shape=[
            jax.ShapeDtypeStruct((B, N, C1), jnp.float32),
            jax.ShapeDtypeStruct((1, C1), jnp.float32),
            jax.ShapeDtypeStruct((1, C1), jnp.float32),
        ],
    )(p1t, pos2, f2t, f1t, w1at, w1bt, b1[None])

    h2, s2, ss2 = pl.pallas_call(
        _k2,
        grid=(B, NBLK),
        in_specs=[
            pl.BlockSpec((1, NB, C1), lambda b, n: (b, n, 0)),
            pl.BlockSpec((1, C1), lambda b, n: (0, 0)),
            pl.BlockSpec((1, C1), lambda b, n: (0, 0)),
            pl.BlockSpec((1, C1), lambda b, n: (0, 0)),
            pl.BlockSpec((1, C1), lambda b, n: (0, 0)),
            pl.BlockSpec((C1, C2), lambda b, n: (0, 0)),
            pl.BlockSpec((1, C2), lambda b, n: (0, 0)),
        ],
        out_specs=[
            pl.BlockSpec((1, NB, C2), lambda b, n: (b, n, 0)),
            pl.BlockSpec((1, C2), lambda b, n: (0, 0)),
            pl.BlockSpec((1, C2), lambda b, n: (0, 0)),
        ],
        out_shape=[
            jax.ShapeDtypeStruct((B, N, C2), jnp.float32),
            jax.ShapeDtypeStruct((1, C2), jnp.float32),
            jax.ShapeDtypeStruct((1, C2), jnp.float32),
        ],
    )(h1, s1, ss1, g1[None], beta1[None], w2t, b2[None])

    out = pl.pallas_call(
        _k3,
        grid=(B, NBLK),
        in_specs=[
            pl.BlockSpec((1, NB, C2), lambda b, n: (b, n, 0)),
            pl.BlockSpec((1, C2), lambda b, n: (0, 0)),
            pl.BlockSpec((1, C2), lambda b, n: (0, 0)),
            pl.BlockSpec((1, C2), lambda b, n: (0, 0)),
            pl.BlockSpec((1, C2), lambda b, n: (0, 0)),
        ],
        out_specs=pl.BlockSpec((1, C2, NB), lambda b, n: (b, 0, n)),
        out_shape=jax.ShapeDtypeStruct((B, C2, N), jnp.float32),
    )(h2, s2, ss2, g2[None], beta2[None])
    return out


# speedup vs baseline: 1.2018x; 1.2018x over previous
import jax
import jax.numpy as jnp
from jax.experimental import pallas as pl

B, N, S, D1, D2 = 4, 8192, 2048, 128, 256
C1, C2 = 256, 128
NB = 2048
NBLK = N // NB
M = B * N


def _k1(p1t_ref, p2_ref, f2t_ref, f1t_ref, w1at_ref, w1bt_ref, b1_ref,
        h1_ref, s_ref, ss_ref):
    b = pl.program_id(0)
    n = pl.program_id(1)
    p1 = p1t_ref[0]
    p2 = p2_ref[0]
    d = (jnp.sum(p1 * p1, axis=1, keepdims=True)
         - 2.0 * jnp.dot(p1, p2, preferred_element_type=jnp.float32)
         + jnp.sum(p2 * p2, axis=0, keepdims=True))
    iota = jax.lax.broadcasted_iota(jnp.int32, (NB, S), 1)

    def extract(dc):
        m = jnp.min(dc, axis=1, keepdims=True)
        i = jnp.min(jnp.where(dc <= m, iota, S), axis=1, keepdims=True)
        dn = jnp.where(iota == i, jnp.inf, dc)
        return m, i, dn

    m1, i1, d = extract(d)
    m2, i2, d = extract(d)
    m3, i3, _ = extract(d)
    r1 = 1.0 / (m1 + 1e-8)
    r2 = 1.0 / (m2 + 1e-8)
    r3 = 1.0 / (m3 + 1e-8)
    nrm = r1 + r2 + r3
    wm = (jnp.where(iota == i1, r1 / nrm, 0.0)
          + jnp.where(iota == i2, r2 / nrm, 0.0)
          + jnp.where(iota == i3, r3 / nrm, 0.0))
    interp = jnp.dot(wm, f2t_ref[0], preferred_element_type=jnp.float32)
    h1 = (jnp.dot(f1t_ref[0], w1at_ref[...], preferred_element_type=jnp.float32)
          + jnp.dot(interp, w1bt_ref[...], preferred_element_type=jnp.float32)
          + b1_ref[...])
    h1_ref[0] = h1

    @pl.when(jnp.logical_and(b == 0, n == 0))
    def _():
        s_ref[...] = jnp.zeros_like(s_ref)
        ss_ref[...] = jnp.zeros_like(ss_ref)

    s_ref[...] += jnp.sum(h1, axis=0, keepdims=True)
    ss_ref[...] += jnp.sum(h1 * h1, axis=0, keepdims=True)


def _k2(h1_ref, s1_ref, ss1_ref, g1_ref, be1_ref, w2t_ref, b2_ref,
        h2_ref, s_ref, ss_ref):
    b = pl.program_id(0)
    n = pl.program_id(1)
    mean = s1_ref[...] / M
    var = ss1_ref[...] / M - mean * mean
    scale = g1_ref[...] * jax.lax.rsqrt(var + 1e-5)
    z = jnp.maximum((h1_ref[0] - mean) * scale + be1_ref[...], 0.0)
    h2 = jnp.dot(z, w2t_ref[...], preferred_element_type=jnp.float32) + b2_ref[...]
    h2_ref[0] = h2

    @pl.when(jnp.logical_and(b == 0, n == 0))
    def _():
        s_ref[...] = jnp.zeros_like(s_ref)
        ss_ref[...] = jnp.zeros_like(ss_ref)

    s_ref[...] += jnp.sum(h2, axis=0, keepdims=True)
    ss_ref[...] += jnp.sum(h2 * h2, axis=0, keepdims=True)


def _k3(h2_ref, s2_ref, ss2_ref, g2_ref, be2_ref, out_ref):
    mean = s2_ref[...] / M
    var = ss2_ref[...] / M - mean * mean
    scale = g2_ref[...] * jax.lax.rsqrt(var + 1e-5)
    z = jnp.maximum((h2_ref[0] - mean) * scale + be2_ref[...], 0.0)
    out_ref[0] = z.T


def kernel(pos1, pos2, feats1, feats2, W1, b1, g1, beta1, W2, b2, g2, beta2):
    p1t = jnp.transpose(pos1, (0, 2, 1))
    f1t = jnp.transpose(feats1, (0, 2, 1))
    f2t = jnp.transpose(feats2, (0, 2, 1))
    w1at = jnp.transpose(W1[:, :D1])
    w1bt = jnp.transpose(W1[:, D1:])
    w2t = jnp.transpose(W2)

    h1, s1, ss1 = pl.pallas_call(
        _k1,
        grid=(B, NBLK),
        in_specs=[
            pl.BlockSpec((1, NB, 3), lambda b, n: (b, n, 0)),
            pl.BlockSpec((1, 3, S), lambda b, n: (b, 0, 0)),
            pl.BlockSpec((1, S, D2), lambda b, n: (b, 0, 0)),
            pl.BlockSpec((1, NB, D1), lambda b, n: (b, n, 0)),
            pl.BlockSpec((D1, C1), lambda b, n: (0, 0)),
            pl.BlockSpec((D2, C1), lambda b, n: (0, 0)),
            pl.BlockSpec((1, C1), lambda b, n: (0, 0)),
        ],
        out_specs=[
            pl.BlockSpec((1, NB, C1), lambda b, n: (b, n, 0)),
            pl.BlockSpec((1, C1), lambda b, n: (0, 0)),
            pl.BlockSpec((1, C1), lambda b, n: (0, 0)),
        ],
        out_shape=[
            jax.ShapeDtypeStruct((B, N, C1), jnp.float32),
            jax.ShapeDtypeStruct((1, C1), jnp.float32),
            jax.ShapeDtypeStruct((1, C1), jnp.float32),
        ],
    )(p1t, pos2, f2t, f1t, w1at, w1bt, b1[None])

    h2, s2, ss2 = pl.pallas_call(
        _k2,
        grid=(B, NBLK),
        in_specs=[
            pl.BlockSpec((1, NB, C1), lambda b, n: (b, n, 0)),
            pl.BlockSpec((1, C1), lambda b, n: (0, 0)),
            pl.BlockSpec((1, C1), lambda b, n: (0, 0)),
            pl.BlockSpec((1, C1), lambda b, n: (0, 0)),
            pl.BlockSpec((1, C1), lambda b, n: (0, 0)),
            pl.BlockSpec((C1, C2), lambda b, n: (0, 0)),
            pl.BlockSpec((1, C2), lambda b, n: (0, 0)),
        ],
        out_specs=[
            pl.BlockSpec((1, NB, C2), lambda b, n: (b, n, 0)),
            pl.BlockSpec((1, C2), lambda b, n: (0, 0)),
            pl.BlockSpec((1, C2), lambda b, n: (0, 0)),
        ],
        out_shape=[
            jax.ShapeDtypeStruct((B, N, C2), jnp.float32),
            jax.ShapeDtypeStruct((1, C2), jnp.float32),
            jax.ShapeDtypeStruct((1, C2), jnp.float32),
        ],
    )(h1, s1, ss1, g1[None], beta1[None], w2t, b2[None])

    out = pl.pallas_call(
        _k3,
        grid=(B, NBLK),
        in_specs=[
            pl.BlockSpec((1, NB, C2), lambda b, n: (b, n, 0)),
            pl.BlockSpec((1, C2), lambda b, n: (0, 0)),
            pl.BlockSpec((1, C2), lambda b, n: (0, 0)),
            pl.BlockSpec((1, C2), lambda b, n: (0, 0)),
            pl.BlockSpec((1, C2), lambda b, n: (0, 0)),
        ],
        out_specs=pl.BlockSpec((1, C2, NB), lambda b, n: (b, 0, n)),
        out_shape=jax.ShapeDtypeStruct((B, C2, N), jnp.float32),
    )(h2, s2, ss2, g2[None], beta2[None])
    return out
